# baseline (speedup 1.0000x reference)
import functools

import jax
import jax.numpy as jnp
from jax import lax
from jax.experimental import pallas as pl
from jax.experimental.pallas import tpu as pltpu

_B, _S, _D = 4, 8192, 768
_N = _B * _S
_BLK = 2048


def _mask_kernel(x_ref, w_ref, gb_ref, hard_ref, cnt_ref):
    i = pl.program_id(0)
    xb = x_ref[...].astype(jnp.bfloat16).astype(jnp.float32)
    wb = w_ref[...].astype(jnp.bfloat16).astype(jnp.float32).reshape(1, _D)
    s = jnp.sum(xb * wb, axis=1, keepdims=True)
    xg = s + gb_ref[...]
    hard = (xg >= 0.0).astype(jnp.float32)
    hard_ref[...] = hard

    @pl.when(i == 0)
    def _init():
        cnt_ref[...] = jnp.zeros_like(cnt_ref)

    cnt_ref[...] += jnp.sum(hard, axis=0, keepdims=True)


def _noise_plus_bias(b):
    eps = 1e-08
    nk1, nk2 = jax.random.split(jax.random.key(1))
    u1 = jax.random.uniform(nk1, (_N, 1), dtype=jnp.float32)
    u2 = jax.random.uniform(nk2, (_N, 1), dtype=jnp.float32)
    g1 = -jnp.log(-jnp.log(u1 + eps) + eps)
    g2 = -jnp.log(-jnp.log(u2 + eps) + eps)
    return g1 - g2 + b[0]


@jax.jit
def kernel(x, W, b):
    x2 = x.reshape(_N, _D)
    gb = _noise_plus_bias(b)
    hard, cnt = pl.pallas_call(
        _mask_kernel,
        grid=(_N // _BLK,),
        in_specs=[
            pl.BlockSpec((_BLK, _D), lambda i: (i, 0)),
            pl.BlockSpec((_D, 1), lambda i: (0, 0)),
            pl.BlockSpec((_BLK, 1), lambda i: (i, 0)),
        ],
        out_specs=[
            pl.BlockSpec((_BLK, 1), lambda i: (i, 0)),
            pl.BlockSpec((1, 1), lambda i: (0, 0)),
        ],
        out_shape=[
            jax.ShapeDtypeStruct((_N, 1), jnp.float32),
            jax.ShapeDtypeStruct((1, 1), jnp.float32),
        ],
    )(x2, W, gb)
    maskloss = (cnt[0, 0] / _N).astype(jnp.float32)
    return hard.reshape(_B, _S, 1), maskloss


# speedup vs baseline: 4.1311x; 4.1311x over previous
import functools

import jax
import jax.numpy as jnp
from jax import lax
from jax.experimental import pallas as pl
from jax.experimental.pallas import tpu as pltpu

_B, _S, _D = 4, 8192, 768
_N = _B * _S
_BLK = 2048
_ROWS = _BLK // 128


def _mask_kernel(x_ref, w_ref, gb_ref, hard_ref, cnt_ref):
    i = pl.program_id(0)
    xb = x_ref[...].astype(jnp.bfloat16).astype(jnp.float32)
    wb = w_ref[...].astype(jnp.bfloat16).astype(jnp.float32).reshape(1, _D)
    s = jnp.sum(xb * wb, axis=1).reshape(_ROWS, 128)
    xg = s + gb_ref[...]
    hard = (xg >= 0.0).astype(jnp.float32)
    hard_ref[...] = hard

    @pl.when(i == 0)
    def _init():
        cnt_ref[...] = jnp.zeros_like(cnt_ref)

    cnt_ref[...] += jnp.sum(hard, axis=0, keepdims=True).reshape(1, 128)


def _noise_plus_bias(b):
    eps = 1e-08
    nk1, nk2 = jax.random.split(jax.random.key(1))
    u1 = jax.random.uniform(nk1, (_N // 128, 128), dtype=jnp.float32)
    u2 = jax.random.uniform(nk2, (_N // 128, 128), dtype=jnp.float32)
    g1 = -jnp.log(-jnp.log(u1 + eps) + eps)
    g2 = -jnp.log(-jnp.log(u2 + eps) + eps)
    return g1 - g2 + b[0]


@jax.jit
def kernel(x, W, b):
    x2 = x.reshape(_N, _D)
    gb = _noise_plus_bias(b)
    hard, cnt = pl.pallas_call(
        _mask_kernel,
        grid=(_N // _BLK,),
        in_specs=[
            pl.BlockSpec((_BLK, _D), lambda i: (i, 0)),
            pl.BlockSpec((_D, 1), lambda i: (0, 0)),
            pl.BlockSpec((_ROWS, 128), lambda i: (i, 0)),
        ],
        out_specs=[
            pl.BlockSpec((_ROWS, 128), lambda i: (i, 0)),
            pl.BlockSpec((1, 128), lambda i: (0, 0)),
        ],
        out_shape=[
            jax.ShapeDtypeStruct((_N // 128, 128), jnp.float32),
            jax.ShapeDtypeStruct((1, 128), jnp.float32),
        ],
    )(x2, W, gb)
    maskloss = (jnp.sum(cnt) / _N).astype(jnp.float32)
    return hard.reshape(_B, _S, 1), maskloss
